# baseline (speedup 1.0000x reference)
import functools
import math

import jax
import jax.numpy as jnp
from jax import lax
from jax.experimental import pallas as pl
from jax.experimental.pallas import tpu as pltpu
from jax.experimental.pallas import tpu_sc as plsc

D = 32
SCALE = math.sqrt(D)
NC, NS = 2, 16
NW = NC * NS
L = 16
CH = 128
NBUF = 4
NJG, NIG = 8, 4


def _make_kernel(NSEQ: int, NB: int):
    jpw = NSEQ // NJG
    ipw = NB // NIG
    qpj = ipw // CH
    nchunk = jpw * qpj
    nouter = nchunk // NBUF

    @functools.partial(
        pl.kernel,
        out_type=jax.ShapeDtypeStruct((NSEQ, D, NB), jnp.float32),
        mesh=plsc.VectorSubcoreMesh(core_axis_name="c", subcore_axis_name="s"),
        scratch_types=[
            pltpu.VMEM((jpw, ipw), jnp.int32),
            pltpu.VMEM((jpw, ipw), jnp.int32),
            [pltpu.VMEM((CH, 2 * D), jnp.float32) for _ in range(NBUF)],
            [pltpu.VMEM((D, CH), jnp.float32) for _ in range(NBUF)],
            [pltpu.SemaphoreType.DMA for _ in range(NBUF)],
            [pltpu.SemaphoreType.DMA for _ in range(NBUF)],
        ],
        compiler_params=pltpu.CompilerParams(
            use_tc_tiling_on_sc=False, needs_layout_passes=False
        ),
    )
    def run(xrow_ref, xpar_ref, t_ref, o_ref, rowslab, parslab, rows, tbuf,
            gsem, osem):
        wid = lax.axis_index("s") * NC + lax.axis_index("c")
        jg = wid // NIG
        ig = wid % NIG

        pltpu.sync_copy(
            xrow_ref.at[pl.ds(jg * jpw, jpw), pl.ds(ig * ipw, ipw)], rowslab
        )
        pltpu.sync_copy(
            xpar_ref.at[pl.ds(jg * jpw, jpw), pl.ds(ig * ipw, ipw)], parslab
        )
        iota = lax.iota(jnp.int32, L)

        def fire(c, k):
            jrow = c // qpj
            qoff = (c % qpj) * CH
            pltpu.async_copy(
                t_ref.at[rowslab.at[jrow, pl.ds(qoff, CH)]],
                rows[k],
                gsem[k],
            )

        def wait_gathers(k):
            pltpu.make_async_copy(t_ref.at[pl.ds(0, CH)], rows[k], gsem[k]).wait()

        def wait_store(k):
            pltpu.make_async_copy(
                tbuf[k], o_ref.at[0, pl.ds(0, D), pl.ds(0, CH)], osem[k]
            ).wait()

        for k in range(NBUF - 1):
            fire(k, k)

        def step(t, carry):
            for k in range(NBUF):
                c = NBUF * t + k
                wait_gathers(k)

                @pl.when(t > 0)
                def _():
                    wait_store(k)

                rv = rows[k]
                tv = tbuf[k]
                jrow = c // qpj
                qoff = (c % qpj) * CH

                @plsc.parallel_loop(0, CH // L)
                def transpose(g):
                    tok = iota + g * L
                    par = parslab[jrow, pl.ds(qoff + g * L, L)]
                    for kk in range(D):
                        v = plsc.load_gather(rv, [tok, par + kk])
                        tv[kk, pl.ds(g * L, L)] = v * SCALE

                pltpu.async_copy(
                    tbuf[k],
                    o_ref.at[
                        jg * jpw + jrow,
                        pl.ds(0, D),
                        pl.ds(ig * ipw + qoff, CH),
                    ],
                    osem[k],
                )

                kb = (k + NBUF - 1) % NBUF
                if k == 0:
                    fire(c + NBUF - 1, kb)
                else:
                    @pl.when(t < nouter - 1)
                    def _():
                        fire(c + NBUF - 1, kb)
            return carry

        lax.fori_loop(0, nouter, step, 0)
        for k in range(NBUF):
            wait_store(k)

    return run


def kernel(x, table):
    NB, NSEQ = x.shape
    xi = x.astype(jnp.int32)
    xrow_t = jnp.transpose(xi >> 1)
    xpar_t = jnp.transpose((xi & 1) << 5)
    t64 = jnp.reshape(table, (table.shape[0] // 2, 2 * D))
    out_t = _make_kernel(NSEQ, NB)(xrow_t, xpar_t, t64)
    return jnp.transpose(out_t, (2, 0, 1))


# speedup vs baseline: 1.2006x; 1.2006x over previous
import functools
import math

import jax
import jax.numpy as jnp
from jax import lax
from jax.experimental import pallas as pl
from jax.experimental.pallas import tpu as pltpu
from jax.experimental.pallas import tpu_sc as plsc

D = 32
SCALE = math.sqrt(D)
NC, NS = 2, 16
NW = NC * NS
GW = 128
NG = 5
CH = NG * GW
NBUF = 4
FBC = 6400


def _feeder(tt, V):
    grid = (V + FBC - 1) // FBC

    def body(in_ref, o_ref):
        t2 = jnp.swapaxes(in_ref[...] * SCALE, 0, 1)
        t3 = jnp.reshape(t2, (FBC // 4, 4, D))
        for c in range(4):
            o_ref[:, pl.ds(c * D, D)] = t3[:, c, :]

    return pl.pallas_call(
        body,
        grid=(grid,),
        in_specs=[pl.BlockSpec((D, FBC), lambda g: (0, g))],
        out_specs=pl.BlockSpec((FBC // 4, 128), lambda g: (g, 0)),
        out_shape=jax.ShapeDtypeStruct((V // 4, 128), jnp.float32),
    )(tt)


def _make_gather(B: int, V: int):
    rows_per_w = B // NW
    nchunk = rows_per_w // CH
    nxrow = rows_per_w // GW
    nouter = nchunk // NBUF

    @functools.partial(
        pl.kernel,
        out_type=jax.ShapeDtypeStruct((B, D), jnp.float32),
        mesh=plsc.VectorSubcoreMesh(core_axis_name="c", subcore_axis_name="s"),
        scratch_types=[
            pltpu.VMEM((nxrow, GW), jnp.int32),
            [pltpu.VMEM((CH, D), jnp.float32) for _ in range(NBUF)],
            [pltpu.SemaphoreType.DMA for _ in range(NBUF)],
            [pltpu.SemaphoreType.DMA for _ in range(NBUF)],
        ],
        compiler_params=pltpu.CompilerParams(
            use_tc_tiling_on_sc=False, needs_layout_passes=False
        ),
    )
    def run(x_ref, t_ref, o_ref, idx_all, rows, gsem, osem):
        wid = lax.axis_index("s") * NC + lax.axis_index("c")
        obase = wid * rows_per_w

        pltpu.sync_copy(x_ref.at[pl.ds(wid * nxrow, nxrow)], idx_all)

        def fire(c, k):
            rb = c * NG
            for j in range(NG):
                pltpu.async_copy(
                    t_ref.at[idx_all.at[rb + j]],
                    rows[k].at[pl.ds(j * GW, GW)],
                    gsem[k],
                )

        def wait_gathers(k):
            pltpu.make_async_copy(o_ref.at[pl.ds(0, CH)], rows[k], gsem[k]).wait()

        def wait_store(k):
            pltpu.make_async_copy(rows[k], o_ref.at[pl.ds(0, CH)], osem[k]).wait()

        for k in range(NBUF - 1):
            fire(k, k)

        def step(t, carry):
            for k in range(NBUF):
                c = NBUF * t + k
                wait_gathers(k)
                pltpu.async_copy(
                    rows[k], o_ref.at[pl.ds(obase + c * CH, CH)], osem[k]
                )
                kb = (k + NBUF - 1) % NBUF
                if k == 0:
                    @pl.when(t > 0)
                    def _():
                        wait_store(kb)
                        fire(c + NBUF - 1, kb)

                    @pl.when(t == 0)
                    def _():
                        fire(c + NBUF - 1, kb)
                else:
                    @pl.when(c + NBUF - 1 < nchunk)
                    def _():
                        wait_store(kb)
                        fire(c + NBUF - 1, kb)
            return carry

        lax.fori_loop(0, nouter, step, 0)
        for k in range(NBUF):
            wait_store(k)

    return run


def kernel(x, table):
    B = x.shape[0] * x.shape[1]
    V = table.shape[0]
    xf = x.reshape(B // GW, GW).astype(jnp.int32)
    tt = jnp.transpose(table)
    tscaled = jnp.reshape(_feeder(tt, V), (V, D))
    out = _make_gather(B, V)(xf, tscaled)
    return out.reshape(x.shape + (D,))


# speedup vs baseline: 1.4158x; 1.1793x over previous
import functools
import math

import jax
import jax.numpy as jnp
from jax import lax
from jax.experimental import pallas as pl
from jax.experimental.pallas import tpu as pltpu
from jax.experimental.pallas import tpu_sc as plsc

D = 32
SCALE = math.sqrt(D)
NC, NS = 2, 16
NW = NC * NS
GW = 128
NG = 5
CH = NG * GW
NBUF = 4
FBC = 6400


def _feeder(tt, V):
    grid = (V + FBC - 1) // FBC

    Q = FBC // 4

    def body(in_ref, o_ref):
        rows = lax.broadcasted_iota(jnp.int32, (D, 4 * D), 0)
        cols = lax.broadcasted_iota(jnp.int32, (D, 4 * D), 1)
        acc = None
        for c in range(4):
            ident_c = jnp.where(
                cols == rows + c * D, jnp.float32(SCALE), jnp.float32(0.0)
            )
            part = lax.dot_general(
                in_ref[:, pl.ds(c * Q, Q)], ident_c, (((0,), (0,)), ((), ())),
                preferred_element_type=jnp.float32,
            )
            acc = part if acc is None else acc + part
        o_ref[...] = acc

    return pl.pallas_call(
        body,
        grid=(grid,),
        in_specs=[pl.BlockSpec((D, FBC), lambda g: (0, g))],
        out_specs=pl.BlockSpec((Q, 128), lambda g: (g, 0)),
        out_shape=jax.ShapeDtypeStruct((grid * Q, 128), jnp.float32),
    )(tt)


def _make_gather(B: int, V: int):
    rows_per_w = B // NW
    nchunk = rows_per_w // CH
    nxrow = rows_per_w // GW
    nouter = nchunk // NBUF

    @functools.partial(
        pl.kernel,
        out_type=jax.ShapeDtypeStruct((B, D), jnp.float32),
        mesh=plsc.VectorSubcoreMesh(core_axis_name="c", subcore_axis_name="s"),
        scratch_types=[
            pltpu.VMEM((nxrow, GW), jnp.int32),
            [pltpu.VMEM((CH, D), jnp.float32) for _ in range(NBUF)],
            [pltpu.SemaphoreType.DMA for _ in range(NBUF)],
            [pltpu.SemaphoreType.DMA for _ in range(NBUF)],
        ],
        compiler_params=pltpu.CompilerParams(
            use_tc_tiling_on_sc=False, needs_layout_passes=False
        ),
    )
    def run(x_ref, t_ref, o_ref, idx_all, rows, gsem, osem):
        wid = lax.axis_index("s") * NC + lax.axis_index("c")
        obase = wid * rows_per_w

        pltpu.sync_copy(x_ref.at[pl.ds(wid * nxrow, nxrow)], idx_all)

        def fire(c, k):
            rb = c * NG
            for j in range(NG):
                pltpu.async_copy(
                    t_ref.at[idx_all.at[rb + j]],
                    rows[k].at[pl.ds(j * GW, GW)],
                    gsem[k],
                )

        def wait_gathers(k):
            pltpu.make_async_copy(o_ref.at[pl.ds(0, CH)], rows[k], gsem[k]).wait()

        def wait_store(k):
            pltpu.make_async_copy(rows[k], o_ref.at[pl.ds(0, CH)], osem[k]).wait()

        for k in range(NBUF - 1):
            fire(k, k)

        def step(t, carry):
            for k in range(NBUF):
                c = NBUF * t + k
                wait_gathers(k)
                pltpu.async_copy(
                    rows[k], o_ref.at[pl.ds(obase + c * CH, CH)], osem[k]
                )
                kb = (k + NBUF - 1) % NBUF
                if k == 0:
                    @pl.when(t > 0)
                    def _():
                        wait_store(kb)
                        fire(c + NBUF - 1, kb)

                    @pl.when(t == 0)
                    def _():
                        fire(c + NBUF - 1, kb)
                else:
                    @pl.when(c + NBUF - 1 < nchunk)
                    def _():
                        wait_store(kb)
                        fire(c + NBUF - 1, kb)
            return carry

        lax.fori_loop(0, nouter, step, 0)
        for k in range(NBUF):
            wait_store(k)

    return run


def kernel(x, table):
    B = x.shape[0] * x.shape[1]
    V = table.shape[0]
    xi = x.astype(jnp.int32)
    rem = xi % FBC
    xr = (xi - rem) + 4 * (rem % (FBC // 4)) + rem // (FBC // 4)
    xf = xr.reshape(B // GW, GW)
    tt = jnp.transpose(table)
    tperm = _feeder(tt, V)
    tscaled = jnp.reshape(tperm, (tperm.shape[0] * 4, D))
    out = _make_gather(B, V)(xf, tscaled)
    return out.reshape(x.shape + (D,))
